# Initial kernel scaffold; baseline (speedup 1.0000x reference)
import jax
import jax.numpy as jnp
from jax.experimental import pallas as pl


def kernel(observation, belief_counts):
    raise NotImplementedError("write your pallas kernel here")


# speedup vs baseline: 2.3777x; 2.3777x over previous
import jax
import jax.numpy as jnp
from jax.experimental import pallas as pl
from jax.experimental.pallas import tpu as pltpu

_DIM = 262144
_NB = 32
_BATCH = 64
_DBLK = 32768
_D4 = _DBLK // 4
_D128 = _DBLK // 128
_GRID = _DIM // _DBLK


def _dot(a, b):
    return jax.lax.dot_general(a, b, (((1,), (0,)), ((), ())),
                               preferred_element_type=jnp.float32)


def _bins_body(obs_ref, bins_ref):
    m = jnp.sum(obs_ref[...], axis=0) * (1.0 / _BATCH)
    nrm = jax.nn.sigmoid(m)
    binsf = jnp.clip(jnp.floor(nrm * (_NB - 1.0)), 0.0, _NB - 1.0)
    bins_ref[...] = binsf.astype(jnp.bfloat16)


def _im_body(cnt_ref, bins_ref, ig_ref, hpri_ref, hpost_ref, mig_ref, ev_ref):
    i = pl.program_id(0)

    c = cnt_ref[...]
    bins_seg = bins_ref[...]

    row_e = jax.lax.broadcasted_iota(jnp.int32, (4, 128), 0)
    lane_e = jax.lax.broadcasted_iota(jnp.int32, (4, 128), 1)
    E = (lane_e // _NB == row_e).astype(jnp.bfloat16)
    lane_b = jax.lax.broadcasted_iota(jnp.int32, (128, 4), 0)
    col_b = jax.lax.broadcasted_iota(jnp.int32, (128, 4), 1)
    B = (lane_b // _NB == col_b).astype(jnp.float32)

    bins_rep = _dot(bins_seg, E)
    L = jnp.log2(jnp.maximum(c, 1e-37))
    lane_f = (jax.lax.broadcasted_iota(jnp.int32, (_D4, 128), 1)
              & (_NB - 1)).astype(jnp.float32)
    oh = jnp.abs(lane_f - bins_rep) < 0.5
    stk = jnp.concatenate([c, c * L, jnp.where(oh, c, 0.0)], axis=0)
    R = _dot(stk, B)

    T = jnp.concatenate([R[:_D4], R[_D4:2 * _D4], R[2 * _D4:]],
                        axis=1).T
    s = T[0:4]
    S1 = T[4:8]
    ch = T[8:12]

    sc = jnp.maximum(s, 1e-8)
    s2 = s + 1.0
    s2c = jnp.maximum(s2, 1e-8)
    ch1 = ch + 1.0
    lg = jnp.log2(jnp.concatenate(
        [sc, s2c, ch1 + 1e-10 * s2c, ch + 1e-10 * sc], axis=0))
    lsc = lg[0:4]
    ls2 = lg[4:8]
    L2h = lg[8:12]
    Lh = lg[12:16]

    hpri_ref[...] = (s * lsc - S1) / sc
    post_sum = S1 - ch * Lh + ch1 * L2h
    hpost_ref[...] = (s2 * ls2 - post_sum) / s2c
    ig = jnp.clip(ch1 / s2c * (L2h - Lh) + (s2 / s2c) * (lsc - ls2),
                  0.0, None)
    ig_ref[...] = ig

    @pl.when(i == 0)
    def _init():
        mig_ref[0] = 0.0

    mig_ref[0] += jnp.sum(ig)

    @pl.when(i == _GRID - 1)
    def _fin():
        mig = mig_ref[0] * (1.0 / _DIM)
        mig_ref[0] = mig
        ev_ref[0] = jax.nn.sigmoid(mig * 50.0 - 1.0)


def kernel(observation, belief_counts):
    obs3 = observation.reshape(_BATCH, _DIM // 128, 128)
    binsf = pl.pallas_call(
        _bins_body,
        grid=(_GRID,),
        in_specs=[pl.BlockSpec((_BATCH, _D128, 128), lambda i: (0, i, 0))],
        out_specs=pl.BlockSpec((_D128, 128), lambda i: (i, 0)),
        out_shape=jax.ShapeDtypeStruct((_DIM // 128, 128), jnp.bfloat16),
    )(obs3)

    cnt2 = belief_counts.reshape(_DIM // 4, 128)
    bins_seg = binsf.reshape(_DIM // 4, 4)
    kvec = jax.ShapeDtypeStruct((4, _DIM // 4), jnp.float32)
    scl = jax.ShapeDtypeStruct((1,), jnp.float32)
    kspec = pl.BlockSpec((4, _D4), lambda i: (0, i))
    ig, hpri, hpost, mig, ev = pl.pallas_call(
        _im_body,
        grid=(_GRID,),
        in_specs=[
            pl.BlockSpec((_D4, 128), lambda i: (i, 0)),
            pl.BlockSpec((_D4, 4), lambda i: (i, 0)),
        ],
        out_specs=(kspec, kspec, kspec,
                   pl.BlockSpec(memory_space=pltpu.SMEM),
                   pl.BlockSpec(memory_space=pltpu.SMEM)),
        out_shape=(kvec, kvec, kvec, scl, scl),
    )(cnt2, bins_seg)
    return (ig.T.reshape(_DIM), mig.reshape(()), hpri.T.reshape(_DIM),
            hpost.T.reshape(_DIM), ev.reshape(()))


# speedup vs baseline: 2.6206x; 1.1021x over previous
import jax
import jax.numpy as jnp
from jax.experimental import pallas as pl
from jax.experimental.pallas import tpu as pltpu

_DIM = 262144
_NB = 32
_BATCH = 64
_DBLK = 32768
_D4 = _DBLK // 4
_D128 = _DBLK // 128
_GRID = _DIM // _DBLK


def _dot(a, b):
    return jax.lax.dot_general(a, b, (((1,), (0,)), ((), ())),
                               preferred_element_type=jnp.float32)


def _bins_body(obs_ref, bins_ref):
    m = jnp.sum(obs_ref[...], axis=0) * (1.0 / _BATCH)
    nrm = jax.nn.sigmoid(m)
    binsf = jnp.clip(jnp.floor(nrm * (_NB - 1.0)), 0.0, _NB - 1.0)
    bins_ref[...] = binsf.astype(jnp.bfloat16)


def _bins(observation):
    return pl.pallas_call(
        _bins_body,
        grid=(_GRID,),
        in_specs=[pl.BlockSpec((_BATCH, _DBLK), lambda i: (0, i))],
        out_specs=pl.BlockSpec((_DBLK,), lambda i: (i,)),
        out_shape=jax.ShapeDtypeStruct((_DIM,), jnp.bfloat16),
    )(observation)


def _im_body(cnt_ref, bins_ref, ig_ref, hpri_ref, hpost_ref, mig_ref, ev_ref):
    i = pl.program_id(0)

    c = cnt_ref[...]
    bins_seg = bins_ref[...]

    row_e = jax.lax.broadcasted_iota(jnp.int32, (4, 128), 0)
    lane_e = jax.lax.broadcasted_iota(jnp.int32, (4, 128), 1)
    E = (lane_e // _NB == row_e).astype(jnp.bfloat16)
    lane_b = jax.lax.broadcasted_iota(jnp.int32, (128, 4), 0)
    col_b = jax.lax.broadcasted_iota(jnp.int32, (128, 4), 1)
    B = (lane_b // _NB == col_b).astype(jnp.float32)

    bins_rep = _dot(bins_seg, E)
    L = jnp.log2(jnp.maximum(c, 1e-37))
    lane_f = (jax.lax.broadcasted_iota(jnp.int32, (_D4, 128), 1)
              & (_NB - 1)).astype(jnp.float32)
    oh = jnp.abs(lane_f - bins_rep) < 0.5
    stk = jnp.concatenate([c, c * L, jnp.where(oh, c, 0.0)], axis=0)
    R = _dot(stk, B)

    T = jnp.concatenate([R[:_D4], R[_D4:2 * _D4], R[2 * _D4:]],
                        axis=1).T
    s = T[0:4]
    S1 = T[4:8]
    ch = T[8:12]

    sc = jnp.maximum(s, 1e-8)
    s2 = s + 1.0
    s2c = jnp.maximum(s2, 1e-8)
    ch1 = ch + 1.0
    lg = jnp.log2(jnp.concatenate(
        [sc, s2c, ch1 + 1e-10 * s2c, ch + 1e-10 * sc], axis=0))
    lsc = lg[0:4]
    ls2 = lg[4:8]
    L2h = lg[8:12]
    Lh = lg[12:16]

    hpri_ref[...] = (s * lsc - S1) / sc
    post_sum = S1 - ch * Lh + ch1 * L2h
    hpost_ref[...] = (s2 * ls2 - post_sum) / s2c
    ig = jnp.clip(ch1 / s2c * (L2h - Lh) + (s2 / s2c) * (lsc - ls2),
                  0.0, None)
    ig_ref[...] = ig

    @pl.when(i == 0)
    def _init():
        mig_ref[0] = 0.0

    mig_ref[0] += jnp.sum(ig)

    @pl.when(i == _GRID - 1)
    def _fin():
        mig = mig_ref[0] * (1.0 / _DIM)
        mig_ref[0] = mig
        ev_ref[0] = jax.nn.sigmoid(mig * 50.0 - 1.0)


def kernel(observation, belief_counts):
    binsf = _bins(observation)
    cnt2 = belief_counts.reshape(_DIM // 4, 128)
    bins_seg = binsf.reshape(_DIM // 4, 4)
    kvec = jax.ShapeDtypeStruct((4, _DIM // 4), jnp.float32)
    scl = jax.ShapeDtypeStruct((1,), jnp.float32)
    kspec = pl.BlockSpec((4, _D4), lambda i: (0, i))
    ig, hpri, hpost, mig, ev = pl.pallas_call(
        _im_body,
        grid=(_GRID,),
        in_specs=[
            pl.BlockSpec((_D4, 128), lambda i: (i, 0)),
            pl.BlockSpec((_D4, 4), lambda i: (i, 0)),
        ],
        out_specs=(kspec, kspec, kspec,
                   pl.BlockSpec(memory_space=pltpu.SMEM),
                   pl.BlockSpec(memory_space=pltpu.SMEM)),
        out_shape=(kvec, kvec, kvec, scl, scl),
    )(cnt2, bins_seg)
    return (ig.T.reshape(_DIM), mig.reshape(()), hpri.T.reshape(_DIM),
            hpost.T.reshape(_DIM), ev.reshape(()))


# speedup vs baseline: 11.7263x; 4.4747x over previous
import jax
import jax.numpy as jnp
from jax.experimental import pallas as pl
from jax.experimental.pallas import tpu as pltpu

_DIM = 262144
_NB = 32
_BATCH = 64
_DBLK = 32768
_GRID = _DIM // _DBLK


def _bins_body(obs_ref, bins_ref):
    m = jnp.sum(obs_ref[...], axis=0) * (1.0 / _BATCH)
    nrm = jax.nn.sigmoid(m)
    binsf = jnp.clip(jnp.floor(nrm * (_NB - 1.0)), 0.0, _NB - 1.0)
    bins_ref[...] = binsf.astype(jnp.bfloat16)


def _bins(observation):
    return pl.pallas_call(
        _bins_body,
        grid=(_GRID,),
        in_specs=[pl.BlockSpec((_BATCH, _DBLK), lambda i: (0, i))],
        out_specs=pl.BlockSpec((_DBLK,), lambda i: (i,)),
        out_shape=jax.ShapeDtypeStruct((_DIM,), jnp.bfloat16),
    )(observation)


def _im_body(cnt_ref, bins_ref, ig_ref, hpri_ref, hpost_ref, mig_ref, ev_ref):
    i = pl.program_id(0)

    ct = cnt_ref[...]
    b1 = bins_ref[...].astype(jnp.float32)

    sub = jax.lax.broadcasted_iota(
        jnp.int32, (_NB, _DBLK), 0).astype(jnp.float32)
    oh = jnp.abs(sub - b1[None, :]) < 0.5

    L = jnp.log2(jnp.maximum(ct, 1e-37))
    s = jnp.sum(ct, axis=0, keepdims=True)
    S1 = jnp.sum(ct * L, axis=0, keepdims=True)
    ch = jnp.sum(jnp.where(oh, ct, 0.0), axis=0, keepdims=True)

    sc = jnp.maximum(s, 1e-8)
    s2 = s + 1.0
    s2c = jnp.maximum(s2, 1e-8)
    ch1 = ch + 1.0
    lg = jnp.log2(jnp.concatenate(
        [sc, s2c, ch1 + 1e-10 * s2c, ch + 1e-10 * sc], axis=0))
    lsc = lg[0:1]
    ls2 = lg[1:2]
    L2h = lg[2:3]
    Lh = lg[3:4]

    hpri_ref[...] = ((s * lsc - S1) / sc)[0]
    post_sum = S1 - ch * Lh + ch1 * L2h
    hpost_ref[...] = ((s2 * ls2 - post_sum) / s2c)[0]
    ig = jnp.clip(ch1 / s2c * (L2h - Lh) + (s2 / s2c) * (lsc - ls2),
                  0.0, None)
    ig_ref[...] = ig[0]

    @pl.when(i == 0)
    def _init():
        mig_ref[0] = 0.0

    mig_ref[0] += jnp.sum(ig)

    @pl.when(i == _GRID - 1)
    def _fin():
        mig = mig_ref[0] * (1.0 / _DIM)
        mig_ref[0] = mig
        ev_ref[0] = jax.nn.sigmoid(mig * 50.0 - 1.0)


def kernel(observation, belief_counts):
    binsf = _bins(observation)
    ct_all = belief_counts.T
    vec = jax.ShapeDtypeStruct((_DIM,), jnp.float32)
    scl = jax.ShapeDtypeStruct((1,), jnp.float32)
    vspec = pl.BlockSpec((_DBLK,), lambda i: (i,))
    ig, hpri, hpost, mig, ev = pl.pallas_call(
        _im_body,
        grid=(_GRID,),
        in_specs=[
            pl.BlockSpec((_NB, _DBLK), lambda i: (0, i)),
            pl.BlockSpec((_DBLK,), lambda i: (i,)),
        ],
        out_specs=(vspec, vspec, vspec,
                   pl.BlockSpec(memory_space=pltpu.SMEM),
                   pl.BlockSpec(memory_space=pltpu.SMEM)),
        out_shape=(vec, vec, vec, scl, scl),
    )(ct_all, binsf)
    return (ig, mig.reshape(()), hpri, hpost, ev.reshape(()))


# speedup vs baseline: 17.1207x; 1.4600x over previous
import jax
import jax.numpy as jnp
from jax.experimental import pallas as pl
from jax.experimental.pallas import tpu as pltpu

_DIM = 262144
_NB = 32
_BATCH = 64
_DBLK = 32768
_GRID = _DIM // _DBLK


def _bins_body(obs_ref, bins_ref):
    m = jnp.sum(obs_ref[...], axis=0) * (1.0 / _BATCH)
    nrm = jax.nn.sigmoid(m)
    binsf = jnp.clip(jnp.floor(nrm * (_NB - 1.0)), 0.0, _NB - 1.0)
    bins_ref[...] = binsf.astype(jnp.bfloat16)


def _bins(observation):
    return pl.pallas_call(
        _bins_body,
        grid=(_GRID,),
        in_specs=[pl.BlockSpec((_BATCH, _DBLK), lambda i: (0, i))],
        out_specs=pl.BlockSpec((_DBLK,), lambda i: (i,)),
        out_shape=jax.ShapeDtypeStruct((_DIM,), jnp.bfloat16),
    )(observation)


def _im_body(cnt_ref, bins_ref, ig_ref, hpri_ref, hpost_ref, mig_ref, ev_ref):
    i = pl.program_id(0)

    ct = cnt_ref[...]
    b1 = bins_ref[...].astype(jnp.float32)

    sub = jax.lax.broadcasted_iota(
        jnp.int32, (_NB, _DBLK), 0).astype(jnp.float32)
    oh = jnp.abs(sub - b1[None, :]) < 0.5

    L = jnp.log2(jnp.maximum(ct, 1e-37))
    ones = jnp.ones((1, _NB), jnp.float32)
    red = lambda x: jax.lax.dot_general(
        ones, x, (((1,), (0,)), ((), ())),
        preferred_element_type=jnp.float32)
    s = red(ct)
    S1 = red(ct * L)
    ch = red(jnp.where(oh, ct, 0.0))

    sc = jnp.maximum(s, 1e-8)
    s2 = s + 1.0
    s2c = jnp.maximum(s2, 1e-8)
    ch1 = ch + 1.0
    lg = jnp.log2(jnp.concatenate(
        [sc, s2c, ch1 + 1e-10 * s2c, ch + 1e-10 * sc], axis=0))
    lsc = lg[0:1]
    ls2 = lg[1:2]
    L2h = lg[2:3]
    Lh = lg[3:4]

    hpri_ref[...] = ((s * lsc - S1) / sc)[0]
    post_sum = S1 - ch * Lh + ch1 * L2h
    hpost_ref[...] = ((s2 * ls2 - post_sum) / s2c)[0]
    ig = jnp.clip(ch1 / s2c * (L2h - Lh) + (s2 / s2c) * (lsc - ls2),
                  0.0, None)
    ig_ref[...] = ig[0]

    @pl.when(i == 0)
    def _init():
        mig_ref[0] = 0.0

    mig_ref[0] += jnp.sum(ig)

    @pl.when(i == _GRID - 1)
    def _fin():
        mig = mig_ref[0] * (1.0 / _DIM)
        mig_ref[0] = mig
        ev_ref[0] = jax.nn.sigmoid(mig * 50.0 - 1.0)


def kernel(observation, belief_counts):
    binsf = _bins(observation)
    ct_all = belief_counts.T
    vec = jax.ShapeDtypeStruct((_DIM,), jnp.float32)
    scl = jax.ShapeDtypeStruct((1,), jnp.float32)
    vspec = pl.BlockSpec((_DBLK,), lambda i: (i,))
    ig, hpri, hpost, mig, ev = pl.pallas_call(
        _im_body,
        grid=(_GRID,),
        in_specs=[
            pl.BlockSpec((_NB, _DBLK), lambda i: (0, i)),
            pl.BlockSpec((_DBLK,), lambda i: (i,)),
        ],
        out_specs=(vspec, vspec, vspec,
                   pl.BlockSpec(memory_space=pltpu.SMEM),
                   pl.BlockSpec(memory_space=pltpu.SMEM)),
        out_shape=(vec, vec, vec, scl, scl),
    )(ct_all, binsf)
    return (ig, mig.reshape(()), hpri, hpost, ev.reshape(()))


# speedup vs baseline: 17.6187x; 1.0291x over previous
import jax
import jax.numpy as jnp
from jax.experimental import pallas as pl
from jax.experimental.pallas import tpu as pltpu

_DIM = 262144
_NB = 32
_BATCH = 64
_DBLK = 32768
_GRID = _DIM // _DBLK


def _bins_body(obs_ref, bins_ref):
    m = jnp.sum(obs_ref[...], axis=0) * (1.0 / _BATCH)
    nrm = jax.nn.sigmoid(m)
    binsf = jnp.clip(jnp.floor(nrm * (_NB - 1.0)), 0.0, _NB - 1.0)
    bins_ref[...] = binsf.astype(jnp.bfloat16)


def _bins(observation):
    return pl.pallas_call(
        _bins_body,
        grid=(_GRID,),
        in_specs=[pl.BlockSpec((_BATCH, _DBLK), lambda i: (0, i))],
        out_specs=pl.BlockSpec((_DBLK,), lambda i: (i,)),
        out_shape=jax.ShapeDtypeStruct((_DIM,), jnp.bfloat16),
    )(observation)


def _im_body(cnt_ref, bins_ref, ig_ref, hpri_ref, hpost_ref, mig_ref, ev_ref):
    i = pl.program_id(0)

    ct = cnt_ref[...]
    b_i = bins_ref[...].astype(jnp.int32)

    sub = jax.lax.broadcasted_iota(jnp.int32, (_NB, _DBLK), 0)
    oh = sub == b_i[None, :]

    L = jnp.log2(jnp.maximum(ct, 1e-37))
    ones = jnp.ones((1, _NB), jnp.float32)
    red = lambda x: jax.lax.dot_general(
        ones, x, (((1,), (0,)), ((), ())),
        preferred_element_type=jnp.float32)
    s = red(ct)
    S1 = red(ct * L)
    ch = red(jnp.where(oh, ct, 0.0))

    sc = jnp.maximum(s, 1e-8)
    s2 = s + 1.0
    s2c = jnp.maximum(s2, 1e-8)
    ch1 = ch + 1.0
    lg = jnp.log2(jnp.concatenate(
        [sc, s2c, ch1 + 1e-10 * s2c, ch + 1e-10 * sc], axis=0))
    lsc = lg[0:1]
    ls2 = lg[1:2]
    L2h = lg[2:3]
    Lh = lg[3:4]

    hpri_ref[...] = ((s * lsc - S1) / sc)[0]
    post_sum = S1 - ch * Lh + ch1 * L2h
    hpost_ref[...] = ((s2 * ls2 - post_sum) / s2c)[0]
    ig = jnp.clip(ch1 / s2c * (L2h - Lh) + (s2 / s2c) * (lsc - ls2),
                  0.0, None)
    ig_ref[...] = ig[0]

    @pl.when(i == 0)
    def _init():
        mig_ref[0] = 0.0

    mig_ref[0] += jnp.sum(ig)

    @pl.when(i == _GRID - 1)
    def _fin():
        mig = mig_ref[0] * (1.0 / _DIM)
        mig_ref[0] = mig
        ev_ref[0] = jax.nn.sigmoid(mig * 50.0 - 1.0)


def kernel(observation, belief_counts):
    binsf = _bins(observation)
    ct_all = belief_counts.T
    vec = jax.ShapeDtypeStruct((_DIM,), jnp.float32)
    scl = jax.ShapeDtypeStruct((1,), jnp.float32)
    vspec = pl.BlockSpec((_DBLK,), lambda i: (i,))
    ig, hpri, hpost, mig, ev = pl.pallas_call(
        _im_body,
        grid=(_GRID,),
        in_specs=[
            pl.BlockSpec((_NB, _DBLK), lambda i: (0, i)),
            pl.BlockSpec((_DBLK,), lambda i: (i,)),
        ],
        out_specs=(vspec, vspec, vspec,
                   pl.BlockSpec(memory_space=pltpu.SMEM),
                   pl.BlockSpec(memory_space=pltpu.SMEM)),
        out_shape=(vec, vec, vec, scl, scl),
    )(ct_all, binsf)
    return (ig, mig.reshape(()), hpri, hpost, ev.reshape(()))


# speedup vs baseline: 22.4404x; 1.2737x over previous
import jax
import jax.numpy as jnp
from jax.experimental import pallas as pl
from jax.experimental.pallas import tpu as pltpu

_DIM = 262144
_NB = 32
_BATCH = 64
_DBLK = 32768
_GRID = _DIM // _DBLK


def _im_body(obs_ref, cnt_ref, ig_ref, hpri_ref, hpost_ref, mig_ref, ev_ref):
    i = pl.program_id(0)

    m = jnp.sum(obs_ref[...], axis=0, keepdims=True) * (1.0 / _BATCH)
    nrm = jax.nn.sigmoid(m)
    b_i = jnp.clip(jnp.floor(nrm * (_NB - 1.0)), 0.0,
                   _NB - 1.0).astype(jnp.int32)

    ct = cnt_ref[...]
    sub = jax.lax.broadcasted_iota(jnp.int32, (_NB, _DBLK), 0)
    oh = sub == b_i

    L = jnp.log2(jnp.maximum(ct, 1e-37))
    ones = jnp.ones((1, _NB), jnp.float32)
    red = lambda x: jax.lax.dot_general(
        ones, x, (((1,), (0,)), ((), ())),
        preferred_element_type=jnp.float32)
    s = red(ct)
    S1 = red(ct * L)
    ch = red(jnp.where(oh, ct, 0.0))

    sc = jnp.maximum(s, 1e-8)
    s2 = s + 1.0
    s2c = jnp.maximum(s2, 1e-8)
    ch1 = ch + 1.0
    lg = jnp.log2(jnp.concatenate(
        [sc, s2c, ch1 + 1e-10 * s2c, ch + 1e-10 * sc], axis=0))
    lsc = lg[0:1]
    ls2 = lg[1:2]
    L2h = lg[2:3]
    Lh = lg[3:4]

    hpri_ref[...] = ((s * lsc - S1) / sc)[0]
    post_sum = S1 - ch * Lh + ch1 * L2h
    hpost_ref[...] = ((s2 * ls2 - post_sum) / s2c)[0]
    ig = jnp.clip(ch1 / s2c * (L2h - Lh) + (s2 / s2c) * (lsc - ls2),
                  0.0, None)
    ig_ref[...] = ig[0]

    @pl.when(i == 0)
    def _init():
        mig_ref[0] = 0.0

    mig_ref[0] += jnp.sum(ig)

    @pl.when(i == _GRID - 1)
    def _fin():
        mig = mig_ref[0] * (1.0 / _DIM)
        mig_ref[0] = mig
        ev_ref[0] = jax.nn.sigmoid(mig * 50.0 - 1.0)


def kernel(observation, belief_counts):
    ct_all = belief_counts.T
    vec = jax.ShapeDtypeStruct((_DIM,), jnp.float32)
    scl = jax.ShapeDtypeStruct((1,), jnp.float32)
    vspec = pl.BlockSpec((_DBLK,), lambda i: (i,))
    ig, hpri, hpost, mig, ev = pl.pallas_call(
        _im_body,
        grid=(_GRID,),
        in_specs=[
            pl.BlockSpec((_BATCH, _DBLK), lambda i: (0, i)),
            pl.BlockSpec((_NB, _DBLK), lambda i: (0, i)),
        ],
        out_specs=(vspec, vspec, vspec,
                   pl.BlockSpec(memory_space=pltpu.SMEM),
                   pl.BlockSpec(memory_space=pltpu.SMEM)),
        out_shape=(vec, vec, vec, scl, scl),
    )(observation, ct_all)
    return (ig, mig.reshape(()), hpri, hpost, ev.reshape(()))
